# baseline (speedup 1.0000x reference)
import functools

import jax
import jax.numpy as jnp
from jax import lax
from jax.experimental import pallas as pl
from jax.experimental.pallas import tpu as pltpu
from jax.experimental.pallas import tpu_sc as plsc

N = 10000
E = 160000
F = 128
T = 4
K = 8
NL = 4
RC = 5.0

BN = 1000
BE = 2000
HI = jax.lax.Precision.HIGHEST


def _phi_body(s_ref, wphi_ref, bphi_ref, o_ref):
    o_ref[0] = (
        jnp.dot(s_ref[...], wphi_ref[0], preferred_element_type=jnp.float32,
                precision=HI)
        + bphi_ref[0]
    )


def _phi(s, Wphi_l, bphi_l):
    return pl.pallas_call(
        _phi_body,
        grid=(T, N // BN),
        in_specs=[
            pl.BlockSpec((BN, F), lambda t, i: (i, 0)),
            pl.BlockSpec((1, F, 3 * F), lambda t, i: (t, 0, 0)),
            pl.BlockSpec((1, 1, 3 * F), lambda t, i: (t, 0, 0)),
        ],
        out_specs=pl.BlockSpec((1, BN, 3 * F), lambda t, i: (t, i, 0)),
        out_shape=jax.ShapeDtypeStruct((T, N, 3 * F), jnp.float32),
    )(s, Wphi_l, bphi_l.reshape(T, 1, 3 * F))


def _msg_body(ps_ref, pd_ref, td_ref, phie_ref, vsrc_ref, wf_ref, ms_ref, mv_ref):
    diff = ps_ref[...] - pd_ref[...]
    d2 = jnp.sum(diff * diff, axis=1, keepdims=True) + 1e-8
    d = jnp.sqrt(d2)
    inv = 1.0 / d
    fc = 0.5 * (jnp.cos(jnp.pi * jnp.clip(d, 0.0, RC) / RC) + 1.0)
    kk = lax.broadcasted_iota(jnp.int32, (BE, K), 1).astype(jnp.float32) + 1.0
    rbf = jnp.sin(kk * (jnp.pi / RC) * d) * inv
    td = td_ref[...]
    w = jnp.zeros((BE, F), jnp.float32)
    for t in range(T):
        wt = jnp.dot(rbf, wf_ref[t], preferred_element_type=jnp.float32,
                     precision=HI)
        w = w + jnp.where(td == t, wt, 0.0)
    w = w * (fc * (1.0 / T))
    phie = phie_ref[...]
    gs, gv, gd = phie[:, :F], phie[:, F:2 * F], phie[:, 2 * F:]
    ms_ref[...] = gs * w
    gvw = gv * w
    gdw = gd * w
    for c in range(3):
        mv_ref[c] = vsrc_ref[c] * gvw + gdw * (diff[:, c:c + 1] * inv)


def _msg(pos_src, pos_dst, tdst2, phie, vsrc, Wf_l):
    return pl.pallas_call(
        _msg_body,
        grid=(E // BE,),
        in_specs=[
            pl.BlockSpec((BE, 3), lambda i: (i, 0)),
            pl.BlockSpec((BE, 3), lambda i: (i, 0)),
            pl.BlockSpec((BE, 1), lambda i: (i, 0)),
            pl.BlockSpec((BE, 3 * F), lambda i: (i, 0)),
            pl.BlockSpec((3, BE, F), lambda i: (0, i, 0)),
            pl.BlockSpec((T, K, F), lambda i: (0, 0, 0)),
        ],
        out_specs=[
            pl.BlockSpec((BE, F), lambda i: (i, 0)),
            pl.BlockSpec((3, BE, F), lambda i: (0, i, 0)),
        ],
        out_shape=[
            jax.ShapeDtypeStruct((E, F), jnp.float32),
            jax.ShapeDtypeStruct((3, E, F), jnp.float32),
        ],
    )(pos_src, pos_dst, tdst2, phie, vsrc, Wf_l)


_B = 80
_EPS = E // 16


def _segsum_sc(dst32, ms, m0, m1, m2, s_in, v0_in, v1_in, v2_in):
    mesh = plsc.VectorSubcoreMesh(core_axis_name="c", subcore_axis_name="s")
    f32 = jnp.float32

    @functools.partial(
        pl.kernel,
        mesh=mesh,
        out_type=[jax.ShapeDtypeStruct((N, F), f32) for _ in range(4)],
        scratch_types=[
            pltpu.VMEM((_B,), jnp.int32),
            pltpu.VMEM((_B, F), f32),
            pltpu.VMEM_SHARED((N, F), f32),
        ],
    )
    def k(dst_hbm, ms_hbm, m0_hbm, m1_hbm, m2_hbm,
          s_hbm, v0_hbm, v1_hbm, v2_hbm,
          so_hbm, o0_hbm, o1_hbm, o2_hbm,
          idx_v, rows_v, acc):
        cid = lax.axis_index("c")
        sid = lax.axis_index("s")

        def one_quantity(src_hbm, init_hbm, out_hbm):
            @pl.when(sid == 0)
            def _():
                pltpu.sync_copy(init_hbm, acc)
            plsc.subcore_barrier()

            def body(i, carry):
                base = sid * _EPS + i * _B
                pltpu.sync_copy(dst_hbm.at[pl.ds(base, _B)], idx_v)
                pltpu.sync_copy(src_hbm.at[pl.ds(base, _B)], rows_v)
                pltpu.sync_copy(rows_v, acc.at[idx_v], add=True)
                return carry

            lax.fori_loop(0, _EPS // _B, body, 0)
            plsc.subcore_barrier()

            @pl.when(sid == 0)
            def _():
                pltpu.sync_copy(acc, out_hbm)
            plsc.subcore_barrier()

        @pl.when(cid == 0)
        def _():
            one_quantity(ms_hbm, s_hbm, so_hbm)
            one_quantity(m0_hbm, v0_hbm, o0_hbm)

        @pl.when(cid == 1)
        def _():
            one_quantity(m1_hbm, v1_hbm, o1_hbm)
            one_quantity(m2_hbm, v2_hbm, o2_hbm)

    return k(dst32, ms, m0, m1, m2, s_in, v0_in, v1_in, v2_in)


def _readout_body(s_ref, w1_ref, b1_ref, w2_ref, b2_ref, o_ref):
    s = s_ref[...]
    part = jnp.sum(s.reshape(N // 8, 8, F), axis=0)
    pooled = jnp.sum(part, axis=0, keepdims=True)
    h = jnp.dot(pooled, w1_ref[...], preferred_element_type=jnp.float32,
                precision=HI)
    h = h + b1_ref[...]
    h = jnp.maximum(h, 0.0) + jnp.log1p(jnp.exp(-jnp.abs(h))) - jnp.log(2.0)
    out = jnp.dot(h, w2_ref[...], preferred_element_type=jnp.float32,
                  precision=HI)
    o_ref[...] = out + b2_ref[...]


def _readout(s, W1, b1, W2, b2):
    return pl.pallas_call(
        _readout_body,
        out_shape=jax.ShapeDtypeStruct((1, 1), jnp.float32),
    )(s, W1, b1.reshape(1, F), W2, b2.reshape(1, 1))


def kernel(atomic_number, edge_index, pos, embed, Wf, Wphi, bphi, W1, b1, W2, b2):
    src, dst = edge_index[0], edge_index[1]
    tdst = atomic_number[dst]
    pos_src = pos[src]
    pos_dst = pos[dst]
    tdst2 = tdst[:, None].astype(jnp.int32)
    dst32 = dst.astype(jnp.int32)
    phi_row = tdst.astype(jnp.int32) * N + src.astype(jnp.int32)

    s = embed[atomic_number]
    v0 = jnp.zeros((N, F), s.dtype)
    v1 = jnp.zeros((N, F), s.dtype)
    v2 = jnp.zeros((N, F), s.dtype)
    for l in range(NL):
        phi = _phi(s, Wphi[l], bphi[l])
        phie = phi.reshape(T * N, 3 * F)[phi_row]
        vsrc = jnp.stack([v0[src], v1[src], v2[src]])
        ms, mv = _msg(pos_src, pos_dst, tdst2, phie, vsrc, Wf[l])
        s, v0, v1, v2 = _segsum_sc(dst32, ms, mv[0], mv[1], mv[2],
                                   s, v0, v1, v2)

    return _readout(s, W1, b1, W2, b2)
